# baseline (speedup 1.0000x reference)
import functools

import jax
import jax.numpy as jnp
from jax.experimental import pallas as pl
from jax.experimental.pallas import tpu as pltpu


def _gcn_body(x_ref, a_ref, w_ref, o_ref, support_ref):
    @pl.when(pl.program_id(0) == 0)
    def _compute_support():
        support_ref[...] = jnp.dot(
            x_ref[...], w_ref[...], preferred_element_type=jnp.float32
        )

    o_ref[...] = jnp.dot(
        a_ref[...], support_ref[...], preferred_element_type=jnp.float32
    )


@jax.jit
def kernel(inputs, adjacency_matrix, W):
    n, d_in = inputs.shape
    d_out = W.shape[1]

    bm = 200 if n % 200 == 0 else n

    return pl.pallas_call(
        _gcn_body,
        grid=(n // bm,),
        in_specs=[
            pl.BlockSpec((n, d_in), lambda i: (0, 0)),
            pl.BlockSpec((bm, n), lambda i: (i, 0)),
            pl.BlockSpec((d_in, d_out), lambda i: (0, 0)),
        ],
        out_specs=pl.BlockSpec((bm, d_out), lambda i: (i, 0)),
        out_shape=jax.ShapeDtypeStruct((n, d_out), jnp.float32),
        scratch_shapes=[pltpu.VMEM((n, d_out), jnp.float32)],
        compiler_params=pltpu.CompilerParams(
            dimension_semantics=("arbitrary",)
        ),
    )(inputs, adjacency_matrix, W)


# speedup vs baseline: 1.0071x; 1.0071x over previous
import functools

import jax
import jax.numpy as jnp
from jax.experimental import pallas as pl
from jax.experimental.pallas import tpu as pltpu


def _gcn_body(x_ref, a_ref, w_ref, o_ref, support_ref):
    @pl.when(pl.program_id(0) == 0)
    def _compute_support():
        support_ref[...] = jnp.dot(
            x_ref[...], w_ref[...], preferred_element_type=jnp.float32
        )

    o_ref[...] = jnp.dot(
        a_ref[...], support_ref[...], preferred_element_type=jnp.float32
    )


@jax.jit
def kernel(inputs, adjacency_matrix, W):
    n, d_in = inputs.shape
    d_out = W.shape[1]

    bm = 400 if n % 400 == 0 else n

    return pl.pallas_call(
        _gcn_body,
        grid=(n // bm,),
        in_specs=[
            pl.BlockSpec((n, d_in), lambda i: (0, 0)),
            pl.BlockSpec((bm, n), lambda i: (i, 0)),
            pl.BlockSpec((d_in, d_out), lambda i: (0, 0)),
        ],
        out_specs=pl.BlockSpec((bm, d_out), lambda i: (i, 0)),
        out_shape=jax.ShapeDtypeStruct((n, d_out), jnp.float32),
        scratch_shapes=[pltpu.VMEM((n, d_out), jnp.float32)],
        compiler_params=pltpu.CompilerParams(
            dimension_semantics=("arbitrary",),
        ),
    )(inputs, adjacency_matrix, W)
